# Initial kernel scaffold; baseline (speedup 1.0000x reference)
import jax
import jax.numpy as jnp
from jax.experimental import pallas as pl


def kernel(xyz1, xyz2, features1, features2, W1, b1, g1, bt1, W2, b2, g2, bt2):
    raise NotImplementedError("write your pallas kernel here")


# speedup vs baseline: 17.4599x; 17.4599x over previous
import functools

import jax
import jax.numpy as jnp
from jax.experimental import pallas as pl


def _k1_body(x1_ref, x2t_ref, f1_ref, f2_ref, w1at_ref, w1bt_ref, b1_ref,
             h1_ref, s1_ref, q1_ref, *, S):
    b = pl.program_id(0)
    n = pl.program_id(1)

    x1 = x1_ref[0]
    Nb = x1.shape[0]
    dx = x1[:, 0:1] - x2t_ref[0, 0:1, :]
    dy = x1[:, 1:2] - x2t_ref[0, 1:2, :]
    dz = x1[:, 2:3] - x2t_ref[0, 2:3, :]
    d2 = dx * dx + dy * dy + dz * dz

    iota = jax.lax.broadcasted_iota(jnp.int32, (Nb, S), 1)
    BIG = jnp.float32(3.4e38)

    ms = []
    ohs = []
    d2w = d2
    for _ in range(3):
        m = jnp.min(d2w, axis=1, keepdims=True)
        idx = jnp.min(jnp.where(d2w == m, iota, S), axis=1, keepdims=True)
        oh = (iota == idx)
        d2w = jnp.where(oh, BIG, d2w)
        ms.append(m)
        ohs.append(oh)

    rs = [1.0 / (jnp.sqrt(jnp.maximum(m, 1e-12)) + 1e-8) for m in ms]
    norm = rs[0] + rs[1] + rs[2]
    wd = (ohs[0] * (rs[0] / norm) + ohs[1] * (rs[1] / norm)
          + ohs[2] * (rs[2] / norm)).astype(jnp.float32)

    interp = jnp.dot(wd, f2_ref[0], preferred_element_type=jnp.float32)

    h1 = (jnp.dot(interp, w1at_ref[...], preferred_element_type=jnp.float32)
          + jnp.dot(f1_ref[0], w1bt_ref[...], preferred_element_type=jnp.float32)
          + b1_ref[...])
    h1_ref[0] = h1

    @pl.when((b == 0) & (n == 0))
    def _():
        s1_ref[...] = jnp.zeros_like(s1_ref)
        q1_ref[...] = jnp.zeros_like(q1_ref)

    s1_ref[...] += jnp.sum(h1, axis=0, keepdims=True)
    q1_ref[...] += jnp.sum(h1 * h1, axis=0, keepdims=True)


def _k2_body(h1_ref, sc_ref, of_ref, w2t_ref, b2_ref, h2_ref, s2_ref, q2_ref):
    b = pl.program_id(0)
    n = pl.program_id(1)
    y = jnp.maximum(h1_ref[0] * sc_ref[...] + of_ref[...], 0.0)
    h2 = jnp.dot(y, w2t_ref[...], preferred_element_type=jnp.float32) + b2_ref[...]
    h2_ref[0] = h2

    @pl.when((b == 0) & (n == 0))
    def _():
        s2_ref[...] = jnp.zeros_like(s2_ref)
        q2_ref[...] = jnp.zeros_like(q2_ref)

    s2_ref[...] += jnp.sum(h2, axis=0, keepdims=True)
    q2_ref[...] += jnp.sum(h2 * h2, axis=0, keepdims=True)


def _k3_body(h2_ref, sc_ref, of_ref, out_ref):
    out_ref[0] = jnp.maximum(h2_ref[0] * sc_ref[...] + of_ref[...], 0.0)


def kernel(xyz1, xyz2, features1, features2, W1, b1, g1, bt1, W2, b2, g2, bt2):
    B, N, _ = xyz1.shape
    S = xyz2.shape[1]
    C1 = features1.shape[-1]
    C2 = features2.shape[-1]
    H = W1.shape[0]
    Nb = 512
    NB = N // Nb
    cnt = jnp.float32(B * N)

    xyz2t = jnp.transpose(xyz2, (0, 2, 1))
    w1at = jnp.transpose(W1[:, :C2])
    w1bt = jnp.transpose(W1[:, C2:])
    w2t = jnp.transpose(W2)

    h1, s1, q1 = pl.pallas_call(
        functools.partial(_k1_body, S=S),
        grid=(B, NB),
        in_specs=[
            pl.BlockSpec((1, Nb, 3), lambda b, n: (b, n, 0)),
            pl.BlockSpec((1, 3, S), lambda b, n: (b, 0, 0)),
            pl.BlockSpec((1, Nb, C1), lambda b, n: (b, n, 0)),
            pl.BlockSpec((1, S, C2), lambda b, n: (b, 0, 0)),
            pl.BlockSpec((C2, H), lambda b, n: (0, 0)),
            pl.BlockSpec((C1, H), lambda b, n: (0, 0)),
            pl.BlockSpec((1, H), lambda b, n: (0, 0)),
        ],
        out_specs=[
            pl.BlockSpec((1, Nb, H), lambda b, n: (b, n, 0)),
            pl.BlockSpec((1, H), lambda b, n: (0, 0)),
            pl.BlockSpec((1, H), lambda b, n: (0, 0)),
        ],
        out_shape=[
            jax.ShapeDtypeStruct((B, N, H), jnp.float32),
            jax.ShapeDtypeStruct((1, H), jnp.float32),
            jax.ShapeDtypeStruct((1, H), jnp.float32),
        ],
    )(xyz1, xyz2t, features1, features2, w1at, w1bt, b1[None, :])

    mean1 = s1 / cnt
    var1 = q1 / cnt - mean1 * mean1
    sc1 = g1[None, :] / jnp.sqrt(var1 + 1e-5)
    of1 = bt1[None, :] - mean1 * sc1

    h2, s2, q2 = pl.pallas_call(
        _k2_body,
        grid=(B, NB),
        in_specs=[
            pl.BlockSpec((1, Nb, H), lambda b, n: (b, n, 0)),
            pl.BlockSpec((1, H), lambda b, n: (0, 0)),
            pl.BlockSpec((1, H), lambda b, n: (0, 0)),
            pl.BlockSpec((H, H), lambda b, n: (0, 0)),
            pl.BlockSpec((1, H), lambda b, n: (0, 0)),
        ],
        out_specs=[
            pl.BlockSpec((1, Nb, H), lambda b, n: (b, n, 0)),
            pl.BlockSpec((1, H), lambda b, n: (0, 0)),
            pl.BlockSpec((1, H), lambda b, n: (0, 0)),
        ],
        out_shape=[
            jax.ShapeDtypeStruct((B, N, H), jnp.float32),
            jax.ShapeDtypeStruct((1, H), jnp.float32),
            jax.ShapeDtypeStruct((1, H), jnp.float32),
        ],
    )(h1, sc1, of1, w2t, b2[None, :])

    mean2 = s2 / cnt
    var2 = q2 / cnt - mean2 * mean2
    sc2 = g2[None, :] / jnp.sqrt(var2 + 1e-5)
    of2 = bt2[None, :] - mean2 * sc2

    out = pl.pallas_call(
        _k3_body,
        grid=(B, NB),
        in_specs=[
            pl.BlockSpec((1, Nb, H), lambda b, n: (b, n, 0)),
            pl.BlockSpec((1, H), lambda b, n: (0, 0)),
            pl.BlockSpec((1, H), lambda b, n: (0, 0)),
        ],
        out_specs=pl.BlockSpec((1, Nb, H), lambda b, n: (b, n, 0)),
        out_shape=jax.ShapeDtypeStruct((B, N, H), jnp.float32),
    )(h2, sc2, of2)

    return out


# speedup vs baseline: 23.0703x; 1.3213x over previous
import functools

import jax
import jax.numpy as jnp
from jax.experimental import pallas as pl


def _k1_body(x1_ref, x2n_ref, b2_ref, f1_ref, f2_ref, w1at_ref, w1bt_ref,
             b1_ref, h1_ref, s1_ref, q1_ref, *, S):
    b = pl.program_id(0)
    n = pl.program_id(1)

    x1 = x1_ref[0]
    d2 = (jnp.dot(x1, x2n_ref[0], preferred_element_type=jnp.float32)
          + b2_ref[0])

    BIG = jnp.float32(3.4e38)
    m1 = jnp.min(d2, axis=1, keepdims=True)
    t = jnp.where(d2 == m1, BIG, d2)
    m2 = jnp.min(t, axis=1, keepdims=True)
    t = jnp.where(t == m2, BIG, t)
    m3 = jnp.min(t, axis=1, keepdims=True)

    a2 = jnp.sum(x1 * x1, axis=1, keepdims=True)
    rs = [1.0 / (jnp.sqrt(jnp.maximum(m + a2, 1e-12)) + 1e-8)
          for m in (m1, m2, m3)]
    norm = rs[0] + rs[1] + rs[2]
    w1, w2, w3 = rs[0] / norm, rs[1] / norm, rs[2] / norm
    zero = jnp.zeros_like(d2)
    wd = jnp.where(d2 == m1, w1,
                   jnp.where(d2 == m2, w2,
                             jnp.where(d2 == m3, w3, zero)))

    interp = jnp.dot(wd, f2_ref[0], preferred_element_type=jnp.float32)

    h1 = (jnp.dot(interp, w1at_ref[...], preferred_element_type=jnp.float32)
          + jnp.dot(f1_ref[0], w1bt_ref[...], preferred_element_type=jnp.float32)
          + b1_ref[...])
    h1_ref[0] = h1

    @pl.when((b == 0) & (n == 0))
    def _():
        s1_ref[...] = jnp.zeros_like(s1_ref)
        q1_ref[...] = jnp.zeros_like(q1_ref)

    s1_ref[...] += jnp.sum(h1, axis=0, keepdims=True)
    q1_ref[...] += jnp.sum(h1 * h1, axis=0, keepdims=True)


def _k2_body(h1_ref, sc_ref, of_ref, w2t_ref, b2_ref, h2_ref, s2_ref, q2_ref):
    b = pl.program_id(0)
    n = pl.program_id(1)
    y = jnp.maximum(h1_ref[0] * sc_ref[...] + of_ref[...], 0.0)
    h2 = jnp.dot(y, w2t_ref[...], preferred_element_type=jnp.float32) + b2_ref[...]
    h2_ref[0] = h2

    @pl.when((b == 0) & (n == 0))
    def _():
        s2_ref[...] = jnp.zeros_like(s2_ref)
        q2_ref[...] = jnp.zeros_like(q2_ref)

    s2_ref[...] += jnp.sum(h2, axis=0, keepdims=True)
    q2_ref[...] += jnp.sum(h2 * h2, axis=0, keepdims=True)


def _k3_body(h2_ref, sc_ref, of_ref, out_ref):
    out_ref[0] = jnp.maximum(h2_ref[0] * sc_ref[...] + of_ref[...], 0.0)


def kernel(xyz1, xyz2, features1, features2, W1, b1, g1, bt1, W2, b2, g2, bt2):
    B, N, _ = xyz1.shape
    S = xyz2.shape[1]
    C1 = features1.shape[-1]
    C2 = features2.shape[-1]
    H = W1.shape[0]
    Nb = 512
    NB = N // Nb
    cnt = jnp.float32(B * N)

    xyz2t = jnp.transpose(xyz2, (0, 2, 1))
    x2n = -2.0 * xyz2t
    b2sq = jnp.sum(xyz2t * xyz2t, axis=1, keepdims=True)
    w1at = jnp.transpose(W1[:, :C2])
    w1bt = jnp.transpose(W1[:, C2:])
    w2t = jnp.transpose(W2)

    h1, s1, q1 = pl.pallas_call(
        functools.partial(_k1_body, S=S),
        grid=(B, NB),
        in_specs=[
            pl.BlockSpec((1, Nb, 3), lambda b, n: (b, n, 0)),
            pl.BlockSpec((1, 3, S), lambda b, n: (b, 0, 0)),
            pl.BlockSpec((1, 1, S), lambda b, n: (b, 0, 0)),
            pl.BlockSpec((1, Nb, C1), lambda b, n: (b, n, 0)),
            pl.BlockSpec((1, S, C2), lambda b, n: (b, 0, 0)),
            pl.BlockSpec((C2, H), lambda b, n: (0, 0)),
            pl.BlockSpec((C1, H), lambda b, n: (0, 0)),
            pl.BlockSpec((1, H), lambda b, n: (0, 0)),
        ],
        out_specs=[
            pl.BlockSpec((1, Nb, H), lambda b, n: (b, n, 0)),
            pl.BlockSpec((1, H), lambda b, n: (0, 0)),
            pl.BlockSpec((1, H), lambda b, n: (0, 0)),
        ],
        out_shape=[
            jax.ShapeDtypeStruct((B, N, H), jnp.float32),
            jax.ShapeDtypeStruct((1, H), jnp.float32),
            jax.ShapeDtypeStruct((1, H), jnp.float32),
        ],
    )(xyz1, x2n, b2sq, features1, features2, w1at, w1bt, b1[None, :])

    mean1 = s1 / cnt
    var1 = q1 / cnt - mean1 * mean1
    sc1 = g1[None, :] / jnp.sqrt(var1 + 1e-5)
    of1 = bt1[None, :] - mean1 * sc1

    h2, s2, q2 = pl.pallas_call(
        _k2_body,
        grid=(B, NB),
        in_specs=[
            pl.BlockSpec((1, Nb, H), lambda b, n: (b, n, 0)),
            pl.BlockSpec((1, H), lambda b, n: (0, 0)),
            pl.BlockSpec((1, H), lambda b, n: (0, 0)),
            pl.BlockSpec((H, H), lambda b, n: (0, 0)),
            pl.BlockSpec((1, H), lambda b, n: (0, 0)),
        ],
        out_specs=[
            pl.BlockSpec((1, Nb, H), lambda b, n: (b, n, 0)),
            pl.BlockSpec((1, H), lambda b, n: (0, 0)),
            pl.BlockSpec((1, H), lambda b, n: (0, 0)),
        ],
        out_shape=[
            jax.ShapeDtypeStruct((B, N, H), jnp.float32),
            jax.ShapeDtypeStruct((1, H), jnp.float32),
            jax.ShapeDtypeStruct((1, H), jnp.float32),
        ],
    )(h1, sc1, of1, w2t, b2[None, :])

    mean2 = s2 / cnt
    var2 = q2 / cnt - mean2 * mean2
    sc2 = g2[None, :] / jnp.sqrt(var2 + 1e-5)
    of2 = bt2[None, :] - mean2 * sc2

    out = pl.pallas_call(
        _k3_body,
        grid=(B, NB),
        in_specs=[
            pl.BlockSpec((1, Nb, H), lambda b, n: (b, n, 0)),
            pl.BlockSpec((1, H), lambda b, n: (0, 0)),
            pl.BlockSpec((1, H), lambda b, n: (0, 0)),
        ],
        out_specs=pl.BlockSpec((1, Nb, H), lambda b, n: (b, n, 0)),
        out_shape=jax.ShapeDtypeStruct((B, N, H), jnp.float32),
    )(h2, sc2, of2)

    return out
